# baseline (speedup 1.0000x reference)
import jax
import jax.numpy as jnp
from jax import lax
from jax.experimental import pallas as pl
from jax.experimental.pallas import tpu as pltpu
from jax.experimental.pallas import tpu_sc as plsc

VOCAB_N = 1000
VOCAB_P = 1024
B_TOTAL = 51200
NC, NS, LANES = 2, 16, 16
NW = NC * NS
BPW = B_TOTAL // NW
CHUNK = 32
NCHUNK = BPW // CHUNK
NBUF = 2
POSCH = 128


def _lse_body(table_ref, lse_ref):
    x = table_ref[...]
    m = jnp.max(x, axis=1, keepdims=True)
    s = jnp.sum(jnp.exp(x - m), axis=1, keepdims=True)
    lse_ref[...] = jnp.log(s) + m


def _loss_body(part_ref, loss_ref):
    loss_ref[...] = jnp.sum(part_ref[...]).reshape(1, 1) * (1.0 / B_TOTAL)


def _gather_body(table_hbm, tflat_hbm, lse_hbm, idx_hbm, pos_hbm,
                 out_hbm, part_hbm,
                 idx_v, pos_v, lse_v, tval_v, acc_v, rows_v,
                 sem_t, sem_g0, sem_g1, sem_w0, sem_w1):
    wid = lax.axis_index("s") * NC + lax.axis_index("c")
    base = wid * BPW
    sems_g = (sem_g0, sem_g1)
    sems_w = (sem_w0, sem_w1)

    pltpu.sync_copy(idx_hbm.at[pl.ds(base, BPW)], idx_v)
    pltpu.sync_copy(pos_hbm.at[pl.ds(base, BPW)], pos_v)
    pltpu.sync_copy(lse_hbm, lse_v)

    loss_copies = []
    offs = [(k * POSCH, POSCH) for k in range(BPW // POSCH)]
    if BPW % POSCH:
        offs.append((BPW - BPW % POSCH, BPW % POSCH))
    for off, ln in offs:
        loss_copies.append(pltpu.make_async_copy(
            tflat_hbm.at[pos_v.at[pl.ds(off, ln)]],
            tval_v.at[pl.ds(off, ln)], sem_t))
    for cp in loss_copies:
        cp.start()

    def gather_copy(c, b):
        return pltpu.make_async_copy(
            table_hbm.at[idx_v.at[pl.ds(c * CHUNK, CHUNK)]],
            rows_v.at[b], sems_g[b])

    def write_copy(c, b):
        return pltpu.make_async_copy(
            rows_v.at[b], out_hbm.at[pl.ds(base + c * CHUNK, CHUNK)],
            sems_w[b])

    for b in range(NBUF):
        gather_copy(b, b).start()

    def outer(i, _):
        for b in range(NBUF):
            c = i * NBUF + b
            gather_copy(c, b).wait()
            write_copy(c, b).start()
            write_copy(c, b).wait()
            gather_copy(c + NBUF, b).start()
        return 0

    lax.fori_loop(0, NCHUNK // NBUF - 1, outer, 0)
    for b in range(NBUF):
        c = NCHUNK - NBUF + b
        gather_copy(c, b).wait()
        write_copy(c, b).start()
        write_copy(c, b).wait()

    for cp in loss_copies:
        cp.wait()
    acc_v[...] = jnp.zeros((LANES,), jnp.float32)

    def group(j, _):
        iv = idx_v[pl.ds(j * LANES, LANES)]
        lse_vals = plsc.load_gather(lse_v, [iv])
        acc_v[...] = acc_v[...] + (lse_vals - tval_v[pl.ds(j * LANES, LANES)])
        return 0

    lax.fori_loop(0, BPW // LANES, group, 0)
    pltpu.sync_copy(acc_v, part_hbm.at[wid])


def kernel(idx, targets, table):
    idx_f = idx.reshape(B_TOTAL).astype(jnp.int32)
    tgt_f = targets.reshape(B_TOTAL).astype(jnp.int32)
    pos_f = idx_f * VOCAB_N + tgt_f
    table_pad = jnp.pad(table, ((0, 0), (0, VOCAB_P - VOCAB_N)))
    table_flat = table.reshape(VOCAB_N * VOCAB_N)

    lse = pl.pallas_call(
        _lse_body,
        out_shape=jax.ShapeDtypeStruct((VOCAB_N, 1), jnp.float32),
    )(table).reshape(VOCAB_N)

    mesh = plsc.VectorSubcoreMesh(
        core_axis_name="c", subcore_axis_name="s",
        num_cores=NC, num_subcores=NS,
    )
    logits_pad, partials = pl.kernel(
        _gather_body,
        out_type=(
            jax.ShapeDtypeStruct((B_TOTAL, VOCAB_P), jnp.float32),
            jax.ShapeDtypeStruct((NW, LANES), jnp.float32),
        ),
        mesh=mesh,
        compiler_params=pltpu.CompilerParams(
            needs_layout_passes=False, use_tc_tiling_on_sc=True
        ),
        scratch_types=(
            pltpu.VMEM((BPW,), jnp.int32),
            pltpu.VMEM((BPW,), jnp.int32),
            pltpu.VMEM((VOCAB_N,), jnp.float32),
            pltpu.VMEM((BPW,), jnp.float32),
            pltpu.VMEM((LANES,), jnp.float32),
            pltpu.VMEM((NBUF, CHUNK, VOCAB_P), jnp.float32),
            pltpu.SemaphoreType.DMA,
            pltpu.SemaphoreType.DMA,
            pltpu.SemaphoreType.DMA,
            pltpu.SemaphoreType.DMA,
            pltpu.SemaphoreType.DMA,
        ),
    )(table_pad, table_flat, lse, idx_f, pos_f)

    logits = lax.slice(logits_pad, (0, 0), (B_TOTAL, VOCAB_N))

    loss = pl.pallas_call(
        _loss_body,
        out_shape=jax.ShapeDtypeStruct((1, 1), jnp.float32),
    )(partials).reshape(())

    return (logits, loss)


# speedup vs baseline: 1.0008x; 1.0008x over previous
import jax
import jax.numpy as jnp
from jax import lax
from jax.experimental import pallas as pl
from jax.experimental.pallas import tpu as pltpu
from jax.experimental.pallas import tpu_sc as plsc

VOCAB_N = 1000
VOCAB_P = 1024
B_TOTAL = 51200
NC, NS, LANES = 2, 16, 16
NW = NC * NS
BPW = B_TOTAL // NW
CHUNK = 32
NCHUNK = BPW // CHUNK
NBUF = 2
POSCH = 128


def _prep_body(table_ref, lse_ref, pad_ref):
    x = table_ref[...]
    m = jnp.max(x, axis=1, keepdims=True)
    s = jnp.sum(jnp.exp(x - m), axis=1, keepdims=True)
    lse_ref[...] = jnp.log(s) + m
    pad_ref[:, :VOCAB_N] = x
    pad_ref[:, VOCAB_N:] = jnp.zeros((VOCAB_N, VOCAB_P - VOCAB_N), jnp.float32)


def _loss_body(part_ref, loss_ref):
    loss_ref[...] = jnp.sum(part_ref[...]).reshape(1, 1) * (1.0 / B_TOTAL)


def _gather_body(table_hbm, tflat_hbm, lse_hbm, idx_hbm, pos_hbm,
                 out_hbm, part_hbm,
                 idx_v, pos_v, lse_v, tval_v, acc_v, rows_v,
                 sem_t, sem_g0, sem_g1, sem_w0, sem_w1):
    wid = lax.axis_index("s") * NC + lax.axis_index("c")
    base = wid * BPW
    sems_g = (sem_g0, sem_g1)
    sems_w = (sem_w0, sem_w1)

    pltpu.sync_copy(idx_hbm.at[pl.ds(base, BPW)], idx_v)
    pltpu.sync_copy(pos_hbm.at[pl.ds(base, BPW)], pos_v)
    pltpu.sync_copy(lse_hbm, lse_v)

    loss_copies = []
    offs = [(k * POSCH, POSCH) for k in range(BPW // POSCH)]
    if BPW % POSCH:
        offs.append((BPW - BPW % POSCH, BPW % POSCH))
    for off, ln in offs:
        loss_copies.append(pltpu.make_async_copy(
            tflat_hbm.at[pos_v.at[pl.ds(off, ln)]],
            tval_v.at[pl.ds(off, ln)], sem_t))
    for cp in loss_copies:
        cp.start()

    def gather_copy(c, b):
        return pltpu.make_async_copy(
            table_hbm.at[idx_v.at[pl.ds(c * CHUNK, CHUNK)]],
            rows_v.at[b], sems_g[b])

    def write_copy(c, b):
        return pltpu.make_async_copy(
            rows_v.at[b], out_hbm.at[pl.ds(base + c * CHUNK, CHUNK)],
            sems_w[b])

    for b in range(NBUF):
        gather_copy(b, b).start()

    for cp in loss_copies:
        cp.wait()
    acc_v[...] = jnp.zeros((LANES,), jnp.float32)

    gpc = (BPW // LANES) // NCHUNK

    def fold_groups(c):
        for k in range(gpc):
            j = c * gpc + k
            iv = idx_v[pl.ds(j * LANES, LANES)]
            lse_vals = plsc.load_gather(lse_v, [iv])
            acc_v[...] = (acc_v[...]
                          + (lse_vals - tval_v[pl.ds(j * LANES, LANES)]))

    def outer(i, _):
        for b in range(NBUF):
            c = i * NBUF + b
            gather_copy(c, b).wait()
            write_copy(c, b).start()
            fold_groups(c)
            write_copy(c, b).wait()
            gather_copy(c + NBUF, b).start()
        return 0

    lax.fori_loop(0, NCHUNK // NBUF - 1, outer, 0)
    for b in range(NBUF):
        c = NCHUNK - NBUF + b
        gather_copy(c, b).wait()
        write_copy(c, b).start()
        fold_groups(c)
        write_copy(c, b).wait()

    pltpu.sync_copy(acc_v, part_hbm.at[wid])


def kernel(idx, targets, table):
    idx_f = idx.reshape(B_TOTAL).astype(jnp.int32)
    tgt_f = targets.reshape(B_TOTAL).astype(jnp.int32)
    pos_f = idx_f * VOCAB_N + tgt_f
    table_flat = table.reshape(VOCAB_N * VOCAB_N)

    lse2d, table_pad = pl.pallas_call(
        _prep_body,
        out_shape=(
            jax.ShapeDtypeStruct((VOCAB_N, 1), jnp.float32),
            jax.ShapeDtypeStruct((VOCAB_N, VOCAB_P), jnp.float32),
        ),
    )(table)
    lse = lse2d.reshape(VOCAB_N)

    mesh = plsc.VectorSubcoreMesh(
        core_axis_name="c", subcore_axis_name="s",
        num_cores=NC, num_subcores=NS,
    )
    logits_pad, partials = pl.kernel(
        _gather_body,
        out_type=(
            jax.ShapeDtypeStruct((B_TOTAL, VOCAB_P), jnp.float32),
            jax.ShapeDtypeStruct((NW, LANES), jnp.float32),
        ),
        mesh=mesh,
        compiler_params=pltpu.CompilerParams(
            needs_layout_passes=False, use_tc_tiling_on_sc=True
        ),
        scratch_types=(
            pltpu.VMEM((BPW,), jnp.int32),
            pltpu.VMEM((BPW,), jnp.int32),
            pltpu.VMEM((VOCAB_N,), jnp.float32),
            pltpu.VMEM((BPW,), jnp.float32),
            pltpu.VMEM((LANES,), jnp.float32),
            pltpu.VMEM((NBUF, CHUNK, VOCAB_P), jnp.float32),
            pltpu.SemaphoreType.DMA,
            pltpu.SemaphoreType.DMA,
            pltpu.SemaphoreType.DMA,
            pltpu.SemaphoreType.DMA,
            pltpu.SemaphoreType.DMA,
        ),
    )(table_pad, table_flat, lse, idx_f, pos_f)

    logits = lax.slice(logits_pad, (0, 0), (B_TOTAL, VOCAB_N))

    loss = pl.pallas_call(
        _loss_body,
        out_shape=jax.ShapeDtypeStruct((1, 1), jnp.float32),
    )(partials).reshape(())

    return (logits, loss)


# speedup vs baseline: 1.0120x; 1.0112x over previous
import jax
import jax.numpy as jnp
from jax import lax
from jax.experimental import pallas as pl
from jax.experimental.pallas import tpu as pltpu
from jax.experimental.pallas import tpu_sc as plsc

VOCAB_N = 1000
VOCAB_P = 1024
B_TOTAL = 51200
NC, NS, LANES = 2, 16, 16
NW = NC * NS
BPW = B_TOTAL // NW
CHUNK = 32
NCHUNK = BPW // CHUNK
NBUF = 2
POSCH = 128


def _lse_body(table_ref, lse_ref):
    x = table_ref[...]
    m = jnp.max(x, axis=1, keepdims=True)
    s = jnp.sum(jnp.exp(x - m), axis=1, keepdims=True)
    lse_ref[...] = jnp.log(s) + m


def _loss_body(part_ref, loss_ref):
    loss_ref[...] = jnp.sum(part_ref[...]).reshape(1, 1) * (1.0 / B_TOTAL)


def _gather_body(table_hbm, tflat_hbm, lse_hbm, idx_hbm, pos_hbm,
                 out_hbm, part_hbm,
                 idx_v, pos_v, lse_v, tval_v, acc_v, rows_v,
                 sem_t, sem_g0, sem_g1, sem_w0, sem_w1):
    wid = lax.axis_index("s") * NC + lax.axis_index("c")
    base = wid * BPW
    sems_g = (sem_g0, sem_g1)
    sems_w = (sem_w0, sem_w1)

    pltpu.sync_copy(idx_hbm.at[pl.ds(base, BPW)], idx_v)
    pltpu.sync_copy(pos_hbm.at[pl.ds(base, BPW)], pos_v)
    pltpu.sync_copy(lse_hbm, lse_v)

    loss_copies = []
    offs = [(k * POSCH, POSCH) for k in range(BPW // POSCH)]
    if BPW % POSCH:
        offs.append((BPW - BPW % POSCH, BPW % POSCH))
    for off, ln in offs:
        loss_copies.append(pltpu.make_async_copy(
            tflat_hbm.at[pos_v.at[pl.ds(off, ln)]],
            tval_v.at[pl.ds(off, ln)], sem_t))
    for cp in loss_copies:
        cp.start()

    def gather_copy(c, b):
        return pltpu.make_async_copy(
            table_hbm.at[idx_v.at[pl.ds(c * CHUNK, CHUNK)]],
            rows_v.at[b], sems_g[b])

    def write_copy(c, b):
        return pltpu.make_async_copy(
            rows_v.at[b], out_hbm.at[pl.ds(base + c * CHUNK, CHUNK)],
            sems_w[b])

    for b in range(NBUF):
        gather_copy(b, b).start()

    for cp in loss_copies:
        cp.wait()
    acc_v[...] = jnp.zeros((LANES,), jnp.float32)

    gpc = (BPW // LANES) // NCHUNK

    def fold_groups(c):
        for k in range(gpc):
            j = c * gpc + k
            iv = idx_v[pl.ds(j * LANES, LANES)]
            lse_vals = plsc.load_gather(lse_v, [iv])
            acc_v[...] = (acc_v[...]
                          + (lse_vals - tval_v[pl.ds(j * LANES, LANES)]))

    def outer(i, _):
        for b in range(NBUF):
            c = i * NBUF + b
            gather_copy(c, b).wait()
            write_copy(c, b).start()
            fold_groups(c)
            write_copy(c, b).wait()
            gather_copy(c + NBUF, b).start()
        return 0

    lax.fori_loop(0, NCHUNK // NBUF - 1, outer, 0)
    for b in range(NBUF):
        c = NCHUNK - NBUF + b
        gather_copy(c, b).wait()
        write_copy(c, b).start()
        fold_groups(c)
        write_copy(c, b).wait()

    pltpu.sync_copy(acc_v, part_hbm.at[wid])


def kernel(idx, targets, table):
    idx_f = idx.reshape(B_TOTAL).astype(jnp.int32)
    tgt_f = targets.reshape(B_TOTAL).astype(jnp.int32)
    pos_f = idx_f * VOCAB_N + tgt_f
    table_pad = jnp.pad(table, ((0, 0), (0, VOCAB_P - VOCAB_N)))
    table_flat = table.reshape(VOCAB_N * VOCAB_N)

    lse = pl.pallas_call(
        _lse_body,
        out_shape=jax.ShapeDtypeStruct((VOCAB_N, 1), jnp.float32),
    )(table).reshape(VOCAB_N)

    mesh = plsc.VectorSubcoreMesh(
        core_axis_name="c", subcore_axis_name="s",
        num_cores=NC, num_subcores=NS,
    )
    logits_pad, partials = pl.kernel(
        _gather_body,
        out_type=(
            jax.ShapeDtypeStruct((B_TOTAL, VOCAB_P), jnp.float32),
            jax.ShapeDtypeStruct((NW, LANES), jnp.float32),
        ),
        mesh=mesh,
        compiler_params=pltpu.CompilerParams(
            needs_layout_passes=False, use_tc_tiling_on_sc=True
        ),
        scratch_types=(
            pltpu.VMEM((BPW,), jnp.int32),
            pltpu.VMEM((BPW,), jnp.int32),
            pltpu.VMEM((VOCAB_N,), jnp.float32),
            pltpu.VMEM((BPW,), jnp.float32),
            pltpu.VMEM((LANES,), jnp.float32),
            pltpu.VMEM((NBUF, CHUNK, VOCAB_P), jnp.float32),
            pltpu.SemaphoreType.DMA,
            pltpu.SemaphoreType.DMA,
            pltpu.SemaphoreType.DMA,
            pltpu.SemaphoreType.DMA,
            pltpu.SemaphoreType.DMA,
        ),
    )(table_pad, table_flat, lse, idx_f, pos_f)

    logits = lax.slice(logits_pad, (0, 0), (B_TOTAL, VOCAB_N))

    loss = pl.pallas_call(
        _loss_body,
        out_shape=jax.ShapeDtypeStruct((1, 1), jnp.float32),
    )(partials).reshape(())

    return (logits, loss)
